# baseline (speedup 1.0000x reference)
import functools

import jax
import jax.numpy as jnp
from jax import lax
from jax.experimental import pallas as pl
from jax.experimental.pallas import tpu as pltpu
from jax.experimental.pallas import tpu_sc as plsc

B = 16384
D = 32
PACK = 4
VROWS = 100001
PROWS = (VROWS + PACK - 1) // PACK

_info = plsc.get_sparse_core_info()
_NC = _info.num_cores
_NS = _info.num_subcores
_NW = _NC * _NS
_BPW = B // _NW
_CHUNK = 128
_NCHUNK = _BPW // _CHUNK

_mesh = plsc.VectorSubcoreMesh(core_axis_name="c", subcore_axis_name="s")


@functools.partial(
    pl.kernel,
    mesh=_mesh,
    out_type=jax.ShapeDtypeStruct((D, B), jnp.float32),
    scratch_types=[
        pltpu.VMEM((_BPW,), jnp.int32),
        pltpu.VMEM((_BPW,), jnp.int32),
        pltpu.VMEM((_BPW,), jnp.int32),
        pltpu.VMEM((_BPW, 128), jnp.float32),
        pltpu.VMEM((D, _BPW), jnp.float32),
        pltpu.SemaphoreType.DMA,
    ],
    compiler_params=pltpu.CompilerParams(needs_layout_passes=False),
)
def _gather_kernel(tp_hbm, idx_hbm, outT_hbm, idx_v, idx4_v, cofs_v,
                   rows_v, outT_v, sem):
    wid = lax.axis_index("s") * _NC + lax.axis_index("c")
    base = wid * _BPW
    pltpu.sync_copy(idx_hbm.at[pl.ds(base, _BPW)], idx_v)

    def split(g, carry):
        vec = idx_v[pl.ds(g * 16, 16)]
        idx4_v[pl.ds(g * 16, 16)] = lax.shift_right_logical(vec, 2)
        cofs_v[pl.ds(g * 16, 16)] = lax.shift_left(
            lax.bitwise_and(vec, 3), 5)
        return carry

    lax.fori_loop(0, _BPW // 16, split, jnp.int32(0), unroll=False)

    copies = [
        pltpu.async_copy(
            tp_hbm.at[idx4_v.at[pl.ds(j * _CHUNK, _CHUNK)]],
            rows_v.at[pl.ds(j * _CHUNK, _CHUNK)],
            sem,
        )
        for j in range(_NCHUNK)
    ]
    for c in copies:
        c.wait()

    iota = lax.iota(jnp.int32, 16)

    def body(g, carry):
        rid = g * 16 + iota
        cof = cofs_v[pl.ds(g * 16, 16)]
        for j in range(D):
            v = plsc.load_gather(rows_v, [rid, cof + j])
            outT_v[j, pl.ds(g * 16, 16)] = v
        return carry

    lax.fori_loop(0, _BPW // 16, body, jnp.int32(0), unroll=False)
    pltpu.sync_copy(outT_v, outT_hbm.at[:, pl.ds(base, _BPW)])


_TCBLK = 512
_TCGRID = (VROWS + _TCBLK - 1) // _TCBLK


def _pack_body(tT_ref, out_ref):
    t = tT_ref[...]
    tt = jnp.transpose(t)
    t3 = tt.reshape(128, PACK, D)
    out_ref[...] = jnp.concatenate(
        [t3[:, a, :] for a in range(PACK)], axis=1)


_pack = pl.pallas_call(
    _pack_body,
    grid=(_TCGRID,),
    in_specs=[pl.BlockSpec((D, _TCBLK), lambda c: (0, c))],
    out_specs=pl.BlockSpec((128, PACK * D), lambda c: (c, 0)),
    out_shape=jax.ShapeDtypeStruct((PROWS, PACK * D), jnp.float32),
)


def kernel(broadcaster, table):
    idx = broadcaster.astype(jnp.int32)
    tp = _pack(table.T)
    outT = _gather_kernel(tp, idx)
    return outT.T


# speedup vs baseline: 2.2739x; 2.2739x over previous
import functools

import jax
import jax.numpy as jnp
from jax import lax
from jax.experimental import pallas as pl
from jax.experimental.pallas import tpu as pltpu
from jax.experimental.pallas import tpu_sc as plsc

B = 16384
D = 32
PACK = 4
VROWS = 100001
_TCBLK = 512
_TCGRID = (VROWS + _TCBLK - 1) // _TCBLK
PROWS = _TCGRID * 128

_info = plsc.get_sparse_core_info()
_NC = _info.num_cores
_NS = _info.num_subcores
_NW = _NC * _NS
_BPW = B // _NW
_CHUNK = 128
_NCHUNK = _BPW // _CHUNK

_mesh = plsc.VectorSubcoreMesh(core_axis_name="c", subcore_axis_name="s")


@functools.partial(
    pl.kernel,
    mesh=_mesh,
    out_type=jax.ShapeDtypeStruct((D, B), jnp.float32),
    scratch_types=[
        pltpu.VMEM((_BPW,), jnp.int32),
        pltpu.VMEM((_BPW,), jnp.int32),
        pltpu.VMEM((_BPW,), jnp.int32),
        pltpu.VMEM((_BPW, 128), jnp.float32),
        pltpu.VMEM((D, _BPW), jnp.float32),
        pltpu.SemaphoreType.DMA,
    ],
    compiler_params=pltpu.CompilerParams(needs_layout_passes=False),
)
def _gather_kernel(tp_hbm, idx_hbm, outT_hbm, idx_v, idx4_v, cofs_v,
                   rows_v, outT_v, sem):
    wid = lax.axis_index("s") * _NC + lax.axis_index("c")
    base = wid * _BPW
    pltpu.sync_copy(idx_hbm.at[pl.ds(base, _BPW)], idx_v)

    def split(g, carry):
        vec = idx_v[pl.ds(g * 16, 16)]
        idx4_v[pl.ds(g * 16, 16)] = lax.bitwise_or(
            lax.shift_left(lax.shift_right_logical(vec, 9), 7),
            lax.bitwise_and(vec, 127))
        cofs_v[pl.ds(g * 16, 16)] = lax.shift_left(
            lax.bitwise_and(lax.shift_right_logical(vec, 7), 3), 5)
        return carry

    lax.fori_loop(0, _BPW // 16, split, jnp.int32(0), unroll=False)

    copies = [
        pltpu.async_copy(
            tp_hbm.at[idx4_v.at[pl.ds(j * _CHUNK, _CHUNK)]],
            rows_v.at[pl.ds(j * _CHUNK, _CHUNK)],
            sem,
        )
        for j in range(_NCHUNK)
    ]
    for c in copies:
        c.wait()

    iota = lax.iota(jnp.int32, 16)

    def body(g, carry):
        rid = g * 16 + iota
        cof = cofs_v[pl.ds(g * 16, 16)]
        for j in range(D):
            v = plsc.load_gather(rows_v, [rid, cof + j])
            outT_v[j, pl.ds(g * 16, 16)] = v
        return carry

    lax.fori_loop(0, _BPW // 16, body, jnp.int32(0), unroll=False)
    pltpu.sync_copy(outT_v, outT_hbm.at[:, pl.ds(base, _BPW)])


_NG = 4
_TCGRID2 = (VROWS + _NG * _TCBLK - 1) // (_NG * _TCBLK)


def _pack_body(tT_ref, out_ref):
    t = tT_ref[...]
    for g in range(_NG):
        tg = t[:, g * 512:(g + 1) * 512]
        s = tg.reshape(D, PACK, 128).swapaxes(0, 1).reshape(128, 128)
        out_ref[g * 128:(g + 1) * 128, :] = jnp.transpose(s)


_pack = pl.pallas_call(
    _pack_body,
    grid=(_TCGRID2,),
    in_specs=[pl.BlockSpec((D, _NG * _TCBLK), lambda c: (0, c))],
    out_specs=pl.BlockSpec((_NG * 128, PACK * D), lambda c: (c, 0)),
    out_shape=jax.ShapeDtypeStruct((PROWS, PACK * D), jnp.float32),
)


def kernel(broadcaster, table):
    idx = broadcaster.astype(jnp.int32)
    tp = _pack(table.T)
    outT = _gather_kernel(tp, idx)
    return outT.T


# speedup vs baseline: 2.4541x; 1.0793x over previous
import functools

import jax
import jax.numpy as jnp
from jax import lax
from jax.experimental import pallas as pl
from jax.experimental.pallas import tpu as pltpu
from jax.experimental.pallas import tpu_sc as plsc

B = 16384
D = 32
PACK = 4
VROWS = 100001
_TCBLK = 512
_TCGRID = (VROWS + _TCBLK - 1) // _TCBLK
PROWS = _TCGRID * 128

_info = plsc.get_sparse_core_info()
_NC = _info.num_cores
_NS = _info.num_subcores
_NW = _NC * _NS
_BPW = B // _NW
_CHUNK = 128
_NCHUNK = _BPW // _CHUNK

_mesh = plsc.VectorSubcoreMesh(core_axis_name="c", subcore_axis_name="s")


@functools.partial(
    pl.kernel,
    mesh=_mesh,
    out_type=jax.ShapeDtypeStruct((D, B), jnp.float32),
    scratch_types=[
        pltpu.VMEM((_BPW,), jnp.int32),
        pltpu.VMEM((_BPW,), jnp.int32),
        pltpu.VMEM((_BPW,), jnp.int32),
        pltpu.VMEM((_BPW, 128), jnp.float32),
        pltpu.VMEM((D, _BPW), jnp.float32),
        pltpu.SemaphoreType.DMA,
    ],
    compiler_params=pltpu.CompilerParams(needs_layout_passes=False),
)
def _gather_kernel(tp_hbm, idx_hbm, outT_hbm, idx_v, idx4_v, foffs_v,
                   rows_v, outT_v, sem):
    wid = lax.axis_index("s") * _NC + lax.axis_index("c")
    base = wid * _BPW
    pltpu.sync_copy(idx_hbm.at[pl.ds(base, _BPW)], idx_v)
    iota = lax.iota(jnp.int32, 16)

    def split(g, carry):
        vec = idx_v[pl.ds(g * 16, 16)]
        idx4_v[pl.ds(g * 16, 16)] = lax.bitwise_or(
            lax.shift_left(lax.shift_right_logical(vec, 9), 7),
            lax.bitwise_and(vec, 127))
        foffs_v[pl.ds(g * 16, 16)] = lax.bitwise_or(
            lax.shift_left(g * 16 + iota, 7),
            lax.shift_left(
                lax.bitwise_and(lax.shift_right_logical(vec, 7), 3), 5))
        return carry

    lax.fori_loop(0, _BPW // 16, split, jnp.int32(0), unroll=False)

    copies = [
        pltpu.async_copy(
            tp_hbm.at[idx4_v.at[pl.ds(j * _CHUNK, _CHUNK)]],
            rows_v.at[pl.ds(j * _CHUNK, _CHUNK)],
            sem,
        )
        for j in range(_NCHUNK)
    ]
    for c in copies:
        c.wait()

    @plsc.parallel_loop(0, _BPW // 16, unroll=2)
    def body(g):
        rid = g * 16 + iota
        cof = lax.bitwise_and(foffs_v[pl.ds(g * 16, 16)], 127)
        for j in range(D):
            v = plsc.load_gather(rows_v, [rid, cof + j])
            outT_v[j, pl.ds(g * 16, 16)] = v
    pltpu.sync_copy(outT_v, outT_hbm.at[:, pl.ds(base, _BPW)])


_NG = 4
_TCGRID2 = (VROWS + _NG * _TCBLK - 1) // (_NG * _TCBLK)


def _pack_body(tT_ref, out_ref):
    t = tT_ref[...]
    for g in range(_NG):
        tg = t[:, g * 512:(g + 1) * 512]
        s = tg.reshape(D, PACK, 128).swapaxes(0, 1).reshape(128, 128)
        out_ref[g * 128:(g + 1) * 128, :] = jnp.transpose(s)


_pack = pl.pallas_call(
    _pack_body,
    grid=(_TCGRID2,),
    in_specs=[pl.BlockSpec((D, _NG * _TCBLK), lambda c: (0, c))],
    out_specs=pl.BlockSpec((_NG * 128, PACK * D), lambda c: (c, 0)),
    out_shape=jax.ShapeDtypeStruct((PROWS, PACK * D), jnp.float32),
)


def kernel(broadcaster, table):
    idx = broadcaster.astype(jnp.int32)
    tp = _pack(table.T)
    outT = _gather_kernel(tp, idx)
    return outT.T


# speedup vs baseline: 2.9556x; 1.2044x over previous
import functools

import jax
import jax.numpy as jnp
from jax import lax
from jax.experimental import pallas as pl
from jax.experimental.pallas import tpu as pltpu
from jax.experimental.pallas import tpu_sc as plsc

B = 16384
D = 32
PACK = 4
VROWS = 100001
_TCBLK = 512
_TCGRID = (VROWS + _TCBLK - 1) // _TCBLK
PROWS = 25600

_info = plsc.get_sparse_core_info()
_NC = _info.num_cores
_NS = _info.num_subcores
_NW = _NC * _NS
_BPW = B // _NW
_CHUNK = 128
_NCHUNK = _BPW // _CHUNK

_mesh = plsc.VectorSubcoreMesh(core_axis_name="c", subcore_axis_name="s")


@functools.partial(
    pl.kernel,
    mesh=_mesh,
    out_type=jax.ShapeDtypeStruct((D, B), jnp.float32),
    scratch_types=[
        pltpu.VMEM((_BPW,), jnp.int32),
        pltpu.VMEM((_BPW,), jnp.int32),
        pltpu.VMEM((_BPW,), jnp.int32),
        pltpu.VMEM((_BPW, 128), jnp.float32),
        pltpu.VMEM((D, _BPW), jnp.float32),
        pltpu.SemaphoreType.DMA,
    ],
    compiler_params=pltpu.CompilerParams(needs_layout_passes=False),
)
def _gather_kernel(tp_hbm, idx_hbm, outT_hbm, idx_v, idx4_v, foffs_v,
                   rows_v, outT_v, sem):
    wid = lax.axis_index("s") * _NC + lax.axis_index("c")
    base = wid * _BPW
    pltpu.sync_copy(idx_hbm.at[pl.ds(base, _BPW)], idx_v)
    iota = lax.iota(jnp.int32, 16)

    def split(g, carry):
        vec = idx_v[pl.ds(g * 16, 16)]
        idx4_v[pl.ds(g * 16, 16)] = lax.bitwise_or(
            lax.shift_left(lax.shift_right_logical(vec, 9), 7),
            lax.bitwise_and(vec, 127))
        foffs_v[pl.ds(g * 16, 16)] = lax.bitwise_or(
            lax.shift_left(g * 16 + iota, 7),
            lax.shift_left(
                lax.bitwise_and(lax.shift_right_logical(vec, 7), 3), 5))
        return carry

    lax.fori_loop(0, _BPW // 16, split, jnp.int32(0), unroll=False)

    copies = [
        pltpu.async_copy(
            tp_hbm.at[idx4_v.at[pl.ds(j * _CHUNK, _CHUNK)]],
            rows_v.at[pl.ds(j * _CHUNK, _CHUNK)],
            sem,
        )
        for j in range(_NCHUNK)
    ]

    _GPC = _CHUNK // 16

    for k in range(_NCHUNK):
        copies[k].wait()

        @plsc.parallel_loop(k * _GPC, (k + 1) * _GPC, unroll=2)
        def body(g):
            rid = g * 16 + iota
            cof = lax.bitwise_and(foffs_v[pl.ds(g * 16, 16)], 127)
            for j in range(D):
                v = plsc.load_gather(rows_v, [rid, cof + j])
                outT_v[j, pl.ds(g * 16, 16)] = v
    pltpu.sync_copy(outT_v, outT_hbm.at[:, pl.ds(base, _BPW)])


_NG = 8
_TCGRID2 = (VROWS + _NG * _TCBLK - 1) // (_NG * _TCBLK)


def _pack_body(tT_ref, out_ref):
    t = tT_ref[...]
    for g in range(_NG):
        tg = t[:, g * 512:(g + 1) * 512]
        s = tg.reshape(D, PACK, 128).swapaxes(0, 1).reshape(128, 128)
        out_ref[g * 128:(g + 1) * 128, :] = jnp.transpose(s)


_pack = pl.pallas_call(
    _pack_body,
    grid=(_TCGRID2,),
    in_specs=[pl.BlockSpec((D, _NG * _TCBLK), lambda c: (0, c))],
    out_specs=pl.BlockSpec((_NG * 128, PACK * D), lambda c: (c, 0)),
    out_shape=jax.ShapeDtypeStruct((PROWS, PACK * D), jnp.float32),
)


def kernel(broadcaster, table):
    idx = broadcaster.astype(jnp.int32)
    tp = _pack(table.T)
    outT = _gather_kernel(tp, idx)
    return outT.T


# speedup vs baseline: 3.1785x; 1.0754x over previous
import functools

import jax
import jax.numpy as jnp
from jax import lax
from jax.experimental import pallas as pl
from jax.experimental.pallas import tpu as pltpu
from jax.experimental.pallas import tpu_sc as plsc

B = 16384
D = 32
PACK = 4
VROWS = 100001
_TCBLK = 512
_TCGRID = (VROWS + _TCBLK - 1) // _TCBLK
PROWS = 26624

_info = plsc.get_sparse_core_info()
_NC = _info.num_cores
_NS = _info.num_subcores
_NW = _NC * _NS
_BPW = B // _NW
_CHUNK = 128
_NCHUNK = _BPW // _CHUNK

_mesh = plsc.VectorSubcoreMesh(core_axis_name="c", subcore_axis_name="s")


@functools.partial(
    pl.kernel,
    mesh=_mesh,
    out_type=jax.ShapeDtypeStruct((D, B), jnp.float32),
    scratch_types=[
        pltpu.VMEM((_BPW,), jnp.int32),
        pltpu.VMEM((_BPW,), jnp.int32),
        pltpu.VMEM((_BPW, 128), jnp.float32),
        pltpu.VMEM((D, _BPW), jnp.float32),
        pltpu.SemaphoreType.DMA,
    ],
    compiler_params=pltpu.CompilerParams(needs_layout_passes=False),
)
def _gather_kernel(tp_hbm, idx4_hbm, cofs_hbm, outT_hbm, idx4_v, cofs_v,
                   rows_v, outT_v, sem):
    wid = lax.axis_index("s") * _NC + lax.axis_index("c")
    base = wid * _BPW
    pltpu.sync_copy(idx4_hbm.at[pl.ds(base, _BPW)], idx4_v)
    pltpu.sync_copy(cofs_hbm.at[pl.ds(base, _BPW)], cofs_v)
    iota = lax.iota(jnp.int32, 16)

    copies = [
        pltpu.async_copy(
            tp_hbm.at[idx4_v.at[pl.ds(j * _CHUNK, _CHUNK)]],
            rows_v.at[pl.ds(j * _CHUNK, _CHUNK)],
            sem,
        )
        for j in range(_NCHUNK)
    ]

    _GPC = _CHUNK // 16

    for k in range(_NCHUNK):
        copies[k].wait()

        @plsc.parallel_loop(k * _GPC, (k + 1) * _GPC, unroll=4)
        def body(g):
            rid = g * 16 + iota
            cof = cofs_v[pl.ds(g * 16, 16)]
            for j in range(D):
                v = plsc.load_gather(rows_v, [rid, cof + j])
                outT_v[j, pl.ds(g * 16, 16)] = v
    pltpu.sync_copy(outT_v, outT_hbm.at[:, pl.ds(base, _BPW)])


_NG = 16
_TCGRID2 = (VROWS + _NG * _TCBLK - 1) // (_NG * _TCBLK)


def _pack_body(tT_ref, out_ref):
    t = tT_ref[...]
    for g in range(_NG):
        tg = t[:, g * 512:(g + 1) * 512]
        s = tg.reshape(D, PACK, 128).swapaxes(0, 1).reshape(128, 128)
        out_ref[g * 128:(g + 1) * 128, :] = jnp.transpose(s)


_pack = pl.pallas_call(
    _pack_body,
    grid=(_TCGRID2,),
    in_specs=[pl.BlockSpec((D, _NG * _TCBLK), lambda c: (0, c))],
    out_specs=pl.BlockSpec((_NG * 128, PACK * D), lambda c: (c, 0)),
    out_shape=jax.ShapeDtypeStruct((PROWS, PACK * D), jnp.float32),
)


def kernel(broadcaster, table):
    idx = broadcaster.astype(jnp.int32)
    idx4 = jnp.bitwise_or(
        jnp.left_shift(jnp.right_shift(idx, 9), 7),
        jnp.bitwise_and(idx, 127))
    cofs = jnp.left_shift(
        jnp.bitwise_and(jnp.right_shift(idx, 7), 3), 5)
    tp = _pack(table.T)
    outT = _gather_kernel(tp, idx4, cofs)
    return outT.T


# speedup vs baseline: 3.4637x; 1.0897x over previous
import functools

import jax
import jax.numpy as jnp
from jax import lax
from jax.experimental import pallas as pl
from jax.experimental.pallas import tpu as pltpu
from jax.experimental.pallas import tpu_sc as plsc

B = 16384
D = 32
PACK = 4
VROWS = 100001
_TCBLK = 512
_TCGRID = (VROWS + _TCBLK - 1) // _TCBLK
PROWS = 28672

_info = plsc.get_sparse_core_info()
_NC = _info.num_cores
_NS = _info.num_subcores
_NW = _NC * _NS
_BPW = B // _NW
_CHUNK = 64
_NCHUNK = _BPW // _CHUNK

_mesh = plsc.VectorSubcoreMesh(core_axis_name="c", subcore_axis_name="s")


@functools.partial(
    pl.kernel,
    mesh=_mesh,
    out_type=jax.ShapeDtypeStruct((D, B), jnp.float32),
    scratch_types=[
        pltpu.VMEM((_BPW,), jnp.int32),
        pltpu.VMEM((_BPW,), jnp.int32),
        pltpu.VMEM((_BPW, 128), jnp.float32),
        pltpu.VMEM((D, _BPW), jnp.float32),
        pltpu.SemaphoreType.DMA,
    ],
    compiler_params=pltpu.CompilerParams(needs_layout_passes=False),
)
def _gather_kernel(tp_hbm, idx4_hbm, cofs_hbm, outT_hbm, idx4_v, cofs_v,
                   rows_v, outT_v, sem):
    wid = lax.axis_index("s") * _NC + lax.axis_index("c")
    base = wid * _BPW
    pltpu.sync_copy(idx4_hbm.at[pl.ds(base, _BPW)], idx4_v)
    pltpu.sync_copy(cofs_hbm.at[pl.ds(base, _BPW)], cofs_v)
    iota = lax.iota(jnp.int32, 16)

    copies = [
        pltpu.async_copy(
            tp_hbm.at[idx4_v.at[pl.ds(j * _CHUNK, _CHUNK)]],
            rows_v.at[pl.ds(j * _CHUNK, _CHUNK)],
            sem,
        )
        for j in range(_NCHUNK)
    ]

    _GPC = _CHUNK // 16

    for k in range(_NCHUNK):
        copies[k].wait()

        @plsc.parallel_loop(k * _GPC, (k + 1) * _GPC, unroll=4)
        def body(g):
            rid = g * 16 + iota
            cof = cofs_v[pl.ds(g * 16, 16)]
            for j in range(D):
                v = plsc.load_gather(rows_v, [rid, cof + j])
                outT_v[j, pl.ds(g * 16, 16)] = v
    pltpu.sync_copy(outT_v, outT_hbm.at[:, pl.ds(base, _BPW)])


_NG = 32
_TCGRID2 = (VROWS + _NG * _TCBLK - 1) // (_NG * _TCBLK)


def _pack_body(tT_ref, out_ref):
    t = tT_ref[...]
    for g in range(_NG):
        tg = t[:, g * 512:(g + 1) * 512]
        s = tg.reshape(D, PACK, 128).swapaxes(0, 1).reshape(128, 128)
        out_ref[g * 128:(g + 1) * 128, :] = jnp.transpose(s)


_pack = pl.pallas_call(
    _pack_body,
    grid=(_TCGRID2,),
    in_specs=[pl.BlockSpec((D, _NG * _TCBLK), lambda c: (0, c))],
    out_specs=pl.BlockSpec((_NG * 128, PACK * D), lambda c: (c, 0)),
    out_shape=jax.ShapeDtypeStruct((PROWS, PACK * D), jnp.float32),
)


def kernel(broadcaster, table):
    idx = broadcaster.astype(jnp.int32)
    idx4 = jnp.bitwise_or(
        jnp.left_shift(jnp.right_shift(idx, 9), 7),
        jnp.bitwise_and(idx, 127))
    cofs = jnp.left_shift(
        jnp.bitwise_and(jnp.right_shift(idx, 7), 3), 5)
    tp = _pack(table.T)
    outT = _gather_kernel(tp, idx4, cofs)
    return outT.T
